# baseline (speedup 1.0000x reference)
import functools

import jax
import jax.numpy as jnp
from jax import lax
from jax.experimental import pallas as pl
from jax.experimental.pallas import tpu as pltpu, tpu_sc as plsc

D = 64
R = 4096
C = 200
SCALE = 8.0

_info = plsc.get_sparse_core_info()
NC, NS, L = _info.num_cores, _info.num_subcores, _info.num_lanes
NW = NC * NS
R_PER_W = R // NW
NBUF = 2


def _sc_embed(x_flat, lut_wide):
    mesh = plsc.VectorSubcoreMesh(core_axis_name="c", subcore_axis_name="s")

    @functools.partial(
        pl.kernel,
        mesh=mesh,
        compiler_params=pltpu.CompilerParams(use_tc_tiling_on_sc=True,
                                             needs_layout_passes=False),
        out_type=jax.ShapeDtypeStruct((R, C, D), jnp.float32),
        scratch_types=(
            [pltpu.VMEM((R_PER_W * C,), jnp.int32)]
            + [pltpu.VMEM((C, 2 * D), jnp.float32) for _ in range(NBUF)]
            + [pltpu.VMEM((1, C, D), jnp.float32) for _ in range(NBUF)]
            + [pltpu.SemaphoreType.DMA for _ in range(2 * NBUF)]
        ),
    )
    def k(idx_hbm, table_hbm, out_hbm, idx_all, *bufs_and_sems):
        pair = bufs_and_sems[:NBUF]
        outb = bufs_and_sems[NBUF:2 * NBUF]
        sg = bufs_and_sems[2 * NBUF:3 * NBUF]
        sw = bufs_and_sems[3 * NBUF:4 * NBUF]

        wid = lax.axis_index("s") * NC + lax.axis_index("c")
        base = wid * R_PER_W
        pltpu.sync_copy(idx_hbm.at[pl.ds(base * C, R_PER_W * C)], idx_all)

        def gather_start(g, b):
            idx_sl = idx_all.at[pl.ds(g * C, C)]
            pltpu.async_copy(table_hbm.at[idx_sl], pair[b], sg[b])

        def gather_wait(g, b):
            idx_sl = idx_all.at[pl.ds(g * C, C)]
            pltpu.make_async_copy(table_hbm.at[idx_sl], pair[b],
                                  sg[b]).wait()

        def wb_start(g, b):
            pltpu.async_copy(outb[b], out_hbm.at[pl.ds(base + g, 1)], sw[b])

        def wb_wait(b):
            pltpu.make_async_copy(outb[b], out_hbm.at[pl.ds(base, 1)],
                                  sw[b]).wait()

        def scale(b):
            def row(r, c):
                for k4 in range(D // L):
                    sl = pl.ds(k4 * L, L)
                    outb[b][0, r, sl] = pair[b][r, sl] * SCALE
                return c
            lax.fori_loop(0, C, row, 0, unroll=8)

        gather_start(0, 0)

        def outer(go, carry):
            for b in range(NBUF):
                g = go * NBUF + b
                nb = (b + 1) % NBUF

                @pl.when(jnp.logical_and(g + 1 < R_PER_W, g >= 1))
                def _():
                    wb_wait(nb)

                @pl.when(g + 1 < R_PER_W)
                def _():
                    gather_start(g + 1, nb)

                gather_wait(g, b)
                scale(b)
                wb_start(g, b)
            return carry

        lax.fori_loop(0, R_PER_W // NBUF, outer, 0)
        for b in range(NBUF):
            wb_wait(b)

    return k(x_flat, lut_wide)


def kernel(x, lut):
    x_flat = x.reshape(-1).astype(jnp.int32)
    lut_wide = jnp.zeros((lut.shape[0], 2 * D), jnp.float32)
    lut_wide = lut_wide.at[:, :D].set(lut)
    return _sc_embed(x_flat, lut_wide)


# speedup vs baseline: 1.3351x; 1.3351x over previous
import functools

import jax
import jax.numpy as jnp
from jax import lax
from jax.experimental import pallas as pl
from jax.experimental.pallas import tpu as pltpu, tpu_sc as plsc

D = 64
R = 4096
C = 200
SCALE = 8.0

_info = plsc.get_sparse_core_info()
NC, NS, L = _info.num_cores, _info.num_subcores, _info.num_lanes
NW = NC * NS
R_PER_W = R // NW
NG = 3
NO = 2


def _sc_embed(x_flat, lut_wide):
    mesh = plsc.VectorSubcoreMesh(core_axis_name="c", subcore_axis_name="s")

    @functools.partial(
        pl.kernel,
        mesh=mesh,
        compiler_params=pltpu.CompilerParams(use_tc_tiling_on_sc=True,
                                             needs_layout_passes=False),
        out_type=jax.ShapeDtypeStruct((R, C, D), jnp.float32),
        scratch_types=(
            [pltpu.VMEM((C,), jnp.int32) for _ in range(NG)]
            + [pltpu.VMEM((C, 2 * D), jnp.float32) for _ in range(NG)]
            + [pltpu.VMEM((1, C, D), jnp.float32) for _ in range(NO)]
            + [pltpu.SemaphoreType.DMA for _ in range(2 * NG + NO)]
        ),
    )
    def k(idx_hbm, table_hbm, out_hbm, *bufs_and_sems):
        idxb = bufs_and_sems[:NG]
        pair = bufs_and_sems[NG:2 * NG]
        outb = bufs_and_sems[2 * NG:2 * NG + NO]
        si = bufs_and_sems[2 * NG + NO:3 * NG + NO]
        sg = bufs_and_sems[3 * NG + NO:4 * NG + NO]
        sw = bufs_and_sems[4 * NG + NO:4 * NG + 2 * NO]

        wid = lax.axis_index("s") * NC + lax.axis_index("c")
        base = wid * R_PER_W

        def idx_start(g, b):
            pltpu.async_copy(idx_hbm.at[pl.ds((base + g) * C, C)], idxb[b],
                             si[b])

        def idx_wait(b):
            pltpu.make_async_copy(idx_hbm.at[pl.ds(base * C, C)], idxb[b],
                                  si[b]).wait()

        def gather_start(b):
            pltpu.async_copy(table_hbm.at[idxb[b]], pair[b], sg[b])

        def gather_wait(b):
            pltpu.make_async_copy(table_hbm.at[idxb[b]], pair[b],
                                  sg[b]).wait()

        def wb_start(g, b):
            pltpu.async_copy(outb[b], out_hbm.at[pl.ds(base + g, 1)], sw[b])

        def wb_wait(b):
            pltpu.make_async_copy(outb[b], out_hbm.at[pl.ds(base, 1)],
                                  sw[b]).wait()

        def scale(gb, ob):
            def row(r, c):
                for k4 in range(D // L):
                    sl = pl.ds(k4 * L, L)
                    outb[ob][0, r, sl] = pair[gb][r, sl] * SCALE
                return c
            lax.fori_loop(0, C, row, 0, unroll=8)

        for b in range(NG):
            idx_start(b, b)
        for b in range(NG - 1):
            idx_wait(b)
            gather_start(b)

        STEP = NG * NO

        def outer(go, carry):
            for j in range(STEP):
                g = go * STEP + j
                b = j % NG
                gn = (b + NG - 1) % NG
                ob = j % NO

                @pl.when(g + NG - 1 < R_PER_W)
                def _():
                    idx_wait(gn)
                    gather_start(gn)

                gather_wait(b)

                @pl.when(g + NG < R_PER_W)
                def _():
                    idx_start(g + NG, b)

                @pl.when(g >= NO)
                def _():
                    wb_wait(ob)

                scale(b, ob)
                wb_start(g, ob)
            return carry

        lax.fori_loop(0, R_PER_W // STEP, outer, 0)

        done = (R_PER_W // STEP) * STEP
        for g in range(done, R_PER_W):
            b = (g - done) % NG
            gather_wait(b)
            ob = (g - done) % NO
            wb_wait(ob)
            scale(b, ob)
            wb_start(g, ob)
        for b in range(NO):
            wb_wait(b)

    return k(x_flat, lut_wide)


def kernel(x, lut):
    x_flat = x.reshape(-1).astype(jnp.int32)
    lut_wide = jnp.pad(lut, ((0, 0), (0, D)))
    return _sc_embed(x_flat, lut_wide)


# speedup vs baseline: 1.3360x; 1.0007x over previous
import functools

import jax
import jax.numpy as jnp
from jax import lax
from jax.experimental import pallas as pl
from jax.experimental.pallas import tpu as pltpu, tpu_sc as plsc

D = 64
R = 4096
C = 200
SCALE = 8.0

_info = plsc.get_sparse_core_info()
NC, NS, L = _info.num_cores, _info.num_subcores, _info.num_lanes
NW = NC * NS
R_PER_W = R // NW
NG = 3
NO = 2


def _sc_embed(x_flat, lut_wide):
    mesh = plsc.VectorSubcoreMesh(core_axis_name="c", subcore_axis_name="s")

    @functools.partial(
        pl.kernel,
        mesh=mesh,
        compiler_params=pltpu.CompilerParams(use_tc_tiling_on_sc=True,
                                             needs_layout_passes=False),
        out_type=jax.ShapeDtypeStruct((R, C, D), jnp.float32),
        scratch_types=(
            [pltpu.VMEM((C,), jnp.int32) for _ in range(NG)]
            + [pltpu.VMEM((C, 2 * D), jnp.float32) for _ in range(NG)]
            + [pltpu.VMEM((1, C, D), jnp.float32) for _ in range(NO)]
            + [pltpu.SemaphoreType.DMA for _ in range(3 * NG + NO)]
        ),
    )
    def k(idx_hbm, table_hbm, out_hbm, *bufs_and_sems):
        idxb = bufs_and_sems[:NG]
        pair = bufs_and_sems[NG:2 * NG]
        outb = bufs_and_sems[2 * NG:2 * NG + NO]
        si = bufs_and_sems[2 * NG + NO:3 * NG + NO]
        sg = bufs_and_sems[3 * NG + NO:4 * NG + NO]
        sg2 = bufs_and_sems[4 * NG + NO:5 * NG + NO]
        sw = bufs_and_sems[5 * NG + NO:5 * NG + 2 * NO]

        wid = lax.axis_index("s") * NC + lax.axis_index("c")
        base = wid * R_PER_W

        def idx_start(g, b):
            pltpu.async_copy(idx_hbm.at[pl.ds((base + g) * C, C)], idxb[b],
                             si[b])

        def idx_wait(b):
            pltpu.make_async_copy(idx_hbm.at[pl.ds(base * C, C)], idxb[b],
                                  si[b]).wait()

        H1, H2 = 96, 104

        def gather_start(b):
            pltpu.async_copy(table_hbm.at[idxb[b].at[pl.ds(0, H1)]],
                             pair[b].at[pl.ds(0, H1)], sg[b])
            pltpu.async_copy(table_hbm.at[idxb[b].at[pl.ds(H1, H2)]],
                             pair[b].at[pl.ds(H1, H2)], sg2[b])

        def gather_wait(b):
            pltpu.make_async_copy(table_hbm.at[idxb[b].at[pl.ds(0, H1)]],
                                  pair[b].at[pl.ds(0, H1)], sg[b]).wait()
            pltpu.make_async_copy(table_hbm.at[idxb[b].at[pl.ds(H1, H2)]],
                                  pair[b].at[pl.ds(H1, H2)], sg2[b]).wait()

        def wb_start(g, b):
            pltpu.async_copy(outb[b], out_hbm.at[pl.ds(base + g, 1)], sw[b])

        def wb_wait(b):
            pltpu.make_async_copy(outb[b], out_hbm.at[pl.ds(base, 1)],
                                  sw[b]).wait()

        def scale(gb, ob):
            def row(r, c):
                for k4 in range(D // L):
                    sl = pl.ds(k4 * L, L)
                    outb[ob][0, r, sl] = pair[gb][r, sl] * SCALE
                return c
            lax.fori_loop(0, C, row, 0, unroll=8)

        for b in range(NG):
            idx_start(b, b)
        for b in range(NG - 1):
            idx_wait(b)
            gather_start(b)

        STEP = NG * NO

        def outer(go, carry):
            for j in range(STEP):
                g = go * STEP + j
                b = j % NG
                gn = (b + NG - 1) % NG
                ob = j % NO

                @pl.when(g + NG - 1 < R_PER_W)
                def _():
                    idx_wait(gn)
                    gather_start(gn)

                gather_wait(b)

                @pl.when(g + NG < R_PER_W)
                def _():
                    idx_start(g + NG, b)

                @pl.when(g >= NO)
                def _():
                    wb_wait(ob)

                scale(b, ob)
                wb_start(g, ob)
            return carry

        lax.fori_loop(0, R_PER_W // STEP, outer, 0)

        done = (R_PER_W // STEP) * STEP
        for g in range(done, R_PER_W):
            b = (g - done) % NG
            gather_wait(b)
            ob = (g - done) % NO
            wb_wait(ob)
            scale(b, ob)
            wb_start(g, ob)
        for b in range(NO):
            wb_wait(b)

    return k(x_flat, lut_wide)


def kernel(x, lut):
    x_flat = x.reshape(-1).astype(jnp.int32)
    lut_wide = jnp.pad(lut, ((0, 0), (0, D)))
    return _sc_embed(x_flat, lut_wide)


# speedup vs baseline: 1.5539x; 1.1631x over previous
import functools

import jax
import jax.numpy as jnp
from jax import lax
from jax.experimental import pallas as pl
from jax.experimental.pallas import tpu as pltpu, tpu_sc as plsc

D = 64
R = 4096
C = 200
SCALE = 8.0

_info = plsc.get_sparse_core_info()
NC, NS, L = _info.num_cores, _info.num_subcores, _info.num_lanes
NW = NC * NS
R_PER_W = R // NW
NG = 3
NO = 2


def _sc_embed(x_flat, lut_wide):
    mesh = plsc.VectorSubcoreMesh(core_axis_name="c", subcore_axis_name="s")

    @functools.partial(
        pl.kernel,
        mesh=mesh,
        compiler_params=pltpu.CompilerParams(use_tc_tiling_on_sc=True,
                                             needs_layout_passes=False),
        out_type=jax.ShapeDtypeStruct((R, C, D), jnp.float32),
        scratch_types=(
            [pltpu.VMEM((C,), jnp.int32) for _ in range(NG)]
            + [pltpu.VMEM((C, 2 * D), jnp.float32) for _ in range(NG)]
            + [pltpu.VMEM((1, C, D), jnp.float32) for _ in range(NO)]
            + [pltpu.SemaphoreType.DMA for _ in range(3 * NG + NO)]
        ),
    )
    def k(idx_hbm, table_hbm, out_hbm, *bufs_and_sems):
        idxb = bufs_and_sems[:NG]
        pair = bufs_and_sems[NG:2 * NG]
        outb = bufs_and_sems[2 * NG:2 * NG + NO]
        si = bufs_and_sems[2 * NG + NO:3 * NG + NO]
        sg = bufs_and_sems[3 * NG + NO:4 * NG + NO]
        sg2 = bufs_and_sems[4 * NG + NO:5 * NG + NO]
        sw = bufs_and_sems[5 * NG + NO:5 * NG + 2 * NO]

        wid = lax.axis_index("s") * NC + lax.axis_index("c")
        base = wid * R_PER_W

        def idx_start(g, b):
            pltpu.async_copy(idx_hbm.at[pl.ds((base + g) * C, C)], idxb[b],
                             si[b])

        def idx_wait(b):
            pltpu.make_async_copy(idx_hbm.at[pl.ds(base * C, C)], idxb[b],
                                  si[b]).wait()

        H1, H2 = 96, 104

        def gather_start(b):
            pltpu.async_copy(table_hbm.at[idxb[b].at[pl.ds(0, H1)]],
                             pair[b].at[pl.ds(0, H1)], sg[b])
            pltpu.async_copy(table_hbm.at[idxb[b].at[pl.ds(H1, H2)]],
                             pair[b].at[pl.ds(H1, H2)], sg2[b])

        def gather_wait(b):
            pltpu.make_async_copy(table_hbm.at[idxb[b].at[pl.ds(0, H1)]],
                                  pair[b].at[pl.ds(0, H1)], sg[b]).wait()
            pltpu.make_async_copy(table_hbm.at[idxb[b].at[pl.ds(H1, H2)]],
                                  pair[b].at[pl.ds(H1, H2)], sg2[b]).wait()

        def wb_start(g, b):
            pltpu.async_copy(outb[b], out_hbm.at[pl.ds(base + g, 1)], sw[b])

        def wb_wait(b):
            pltpu.make_async_copy(outb[b], out_hbm.at[pl.ds(base, 1)],
                                  sw[b]).wait()

        def scale(gb, ob):
            @plsc.parallel_loop(0, C, unroll=8)
            def _row(r):
                for k4 in range(D // L):
                    sl = pl.ds(k4 * L, L)
                    outb[ob][0, r, sl] = pair[gb][r, sl] * SCALE

        for b in range(NG):
            idx_start(b, b)
        for b in range(NG - 1):
            idx_wait(b)
            gather_start(b)

        STEP = NG * NO

        def outer(go, carry):
            for j in range(STEP):
                g = go * STEP + j
                b = j % NG
                gn = (b + NG - 1) % NG
                ob = j % NO

                @pl.when(g + NG - 1 < R_PER_W)
                def _():
                    idx_wait(gn)
                    gather_start(gn)

                gather_wait(b)

                @pl.when(g + NG < R_PER_W)
                def _():
                    idx_start(g + NG, b)

                @pl.when(g >= NO)
                def _():
                    wb_wait(ob)

                scale(b, ob)
                wb_start(g, ob)
            return carry

        lax.fori_loop(0, R_PER_W // STEP, outer, 0)

        done = (R_PER_W // STEP) * STEP
        for g in range(done, R_PER_W):
            b = (g - done) % NG
            gather_wait(b)
            ob = (g - done) % NO
            wb_wait(ob)
            scale(b, ob)
            wb_start(g, ob)
        for b in range(NO):
            wb_wait(b)

    return k(x_flat, lut_wide)


def kernel(x, lut):
    x_flat = x.reshape(-1).astype(jnp.int32)
    lut_wide = jnp.pad(lut, ((0, 0), (0, D)))
    return _sc_embed(x_flat, lut_wide)
